# Initial kernel scaffold; baseline (speedup 1.0000x reference)
import jax
import jax.numpy as jnp
from jax.experimental import pallas as pl


def kernel(x, router_W, router_temp, W1, b1, W2, b2):
    raise NotImplementedError("write your pallas kernel here")


# speedup vs baseline: 1.7333x; 1.7333x over previous
import functools
import jax
import jax.numpy as jnp
from jax.experimental import pallas as pl
from jax.experimental.pallas import tpu as pltpu

_E = 8
_K = 2
_D = 768
_DFF = 2048
_FB = 512
_NF = _DFF // _FB


def _router_body(x_ref, rw_ref, temp_ref, cw_ref):
    x = x_ref[...]
    logits = jax.lax.dot_general(
        x, rw_ref[...], (((1,), (1,)), ((), ())),
        preferred_element_type=jnp.float32)
    logits = logits / temp_ref[...]
    t_rows = logits.shape[0]
    lane = jax.lax.broadcasted_iota(jnp.int32, (t_rows, _E), 1)
    m0 = jnp.max(logits, axis=1, keepdims=True)
    e0 = jnp.min(jnp.where(logits == m0, lane, _E), axis=1, keepdims=True)
    masked = jnp.where(lane == e0, -jnp.inf, logits)
    m1 = jnp.max(masked, axis=1, keepdims=True)
    e1 = jnp.min(jnp.where(masked == m1, lane, _E), axis=1, keepdims=True)
    w0 = 1.0 / (1.0 + jnp.exp(m1 - m0))
    w1 = 1.0 - w0
    cw_ref[...] = jnp.where(lane == e0, w0, jnp.where(lane == e1, w1, 0.0))


def _ffn_body(cw_ref, x_ref, w1a_ref, w1g_ref, b1a_ref, b1g_ref, w2_ref,
              b2_ref, out_ref):
    e = pl.program_id(0)
    f = pl.program_id(1)

    @pl.when((e == 0) & (f == 0))
    def _():
        out_ref[...] = jnp.zeros_like(out_ref)

    x = x_ref[...]
    a = jnp.dot(x, w1a_ref[0], preferred_element_type=jnp.float32)
    a = a + b1a_ref[0]
    g = jnp.dot(x, w1g_ref[0], preferred_element_type=jnp.float32)
    g = g + b1g_ref[0]
    act = a * (g / (1.0 + jnp.exp(-g)))
    y = jnp.dot(act, w2_ref[0], preferred_element_type=jnp.float32)
    y = jnp.where(f == 0, y + b2_ref[0], y)
    lane = jax.lax.broadcasted_iota(jnp.int32, (x.shape[0], _E), 1)
    cwcol = jnp.sum(jnp.where(lane == e, cw_ref[...], 0.0), axis=1,
                    keepdims=True)
    out_ref[...] += y * cwcol


def kernel(x, router_W, router_temp, W1, b1, W2, b2):
    B, S, D = x.shape
    x_flat = x.reshape(-1, D)
    T = x_flat.shape[0]

    cw = pl.pallas_call(
        _router_body,
        out_shape=jax.ShapeDtypeStruct((T, _E), jnp.float32),
    )(x_flat, router_W, router_temp.reshape(1, 1))

    out = pl.pallas_call(
        _ffn_body,
        grid=(_E, _NF),
        in_specs=[
            pl.BlockSpec((T, _E), lambda e, f: (0, 0)),
            pl.BlockSpec((T, D), lambda e, f: (0, 0)),
            pl.BlockSpec((1, D, _FB), lambda e, f: (e, 0, f)),
            pl.BlockSpec((1, D, _FB), lambda e, f: (e, 0, f + _NF)),
            pl.BlockSpec((1, 1, _FB), lambda e, f: (e, 0, f)),
            pl.BlockSpec((1, 1, _FB), lambda e, f: (e, 0, f + _NF)),
            pl.BlockSpec((1, _FB, D), lambda e, f: (e, f, 0)),
            pl.BlockSpec((1, 1, D), lambda e, f: (e, 0, 0)),
        ],
        out_specs=pl.BlockSpec((T, D), lambda e, f: (0, 0)),
        out_shape=jax.ShapeDtypeStruct((T, D), jnp.float32),
        compiler_params=pltpu.CompilerParams(
            dimension_semantics=("arbitrary", "arbitrary")),
    )(cw, x_flat, W1, W1, b1.reshape(_E, 1, 2 * _DFF),
      b1.reshape(_E, 1, 2 * _DFF), W2, b2.reshape(_E, 1, D))

    return out.reshape(B, S, D)


# speedup vs baseline: 2.0881x; 1.2047x over previous
import functools
import jax
import jax.numpy as jnp
from jax import lax
from jax.experimental import pallas as pl
from jax.experimental.pallas import tpu as pltpu

_E = 8
_K = 2
_D = 768
_DFF = 2048
_TILE = 128


def _incl_cumsum_ax0(v):
    s = 1
    while s < v.shape[0]:
        v = v + jnp.concatenate(
            [jnp.zeros((s, v.shape[1]), v.dtype), v[:-s]], axis=0)
        s *= 2
    return v


def _router_body(x_ref, rw_ref, temp_ref, pos0_ref, pos1_ref, w0_ref, w1_ref,
                 te_ref):
    nt = te_ref.shape[0]
    x = x_ref[...]
    logits = jax.lax.dot_general(
        x, rw_ref[...], (((1,), (1,)), ((), ())),
        preferred_element_type=jnp.float32)
    logits = logits / temp_ref[...]
    t_rows = logits.shape[0]
    lane = jax.lax.broadcasted_iota(jnp.int32, (t_rows, _E), 1)
    m0 = jnp.max(logits, axis=1, keepdims=True)
    e0 = jnp.min(jnp.where(logits == m0, lane, _E), axis=1, keepdims=True)
    masked = jnp.where(lane == e0, -jnp.inf, logits)
    m1 = jnp.max(masked, axis=1, keepdims=True)
    e1 = jnp.min(jnp.where(masked == m1, lane, _E), axis=1, keepdims=True)
    w0_ref[...] = 1.0 / (1.0 + jnp.exp(m1 - m0))
    w1_ref[...] = 1.0 - w0_ref[...]

    oh0 = (lane == e0).astype(jnp.int32)
    oh1 = (lane == e1).astype(jnp.int32)
    c0 = _incl_cumsum_ax0(oh0)
    c1 = _incl_cumsum_ax0(oh1)
    total0 = c0[t_rows - 1:t_rows, :]
    total1 = c1[t_rows - 1:t_rows, :]
    counts = total0 + total1
    pt = (counts + (_TILE - 1)) // _TILE
    ltri = (jax.lax.broadcasted_iota(jnp.int32, (_E, _E), 0)
            < jax.lax.broadcasted_iota(jnp.int32, (_E, _E), 1))
    ts = jnp.dot(pt.astype(jnp.float32), ltri.astype(jnp.float32),
                 preferred_element_type=jnp.float32).astype(jnp.int32)
    ao = ts * _TILE
    rank0 = jnp.sum(oh0 * c0, axis=1, keepdims=True) - 1
    rank1 = (jnp.sum(oh1 * c1, axis=1, keepdims=True) - 1
             + jnp.sum(oh1 * total0, axis=1, keepdims=True))
    pos0_ref[...] = jnp.sum(oh0 * ao, axis=1, keepdims=True) + rank0
    pos1_ref[...] = jnp.sum(oh1 * ao, axis=1, keepdims=True) + rank1
    rowi = jax.lax.broadcasted_iota(jnp.int32, (nt, _E), 0)
    te_ref[...] = jnp.sum((rowi >= ts).astype(jnp.int32), axis=1,
                          keepdims=True) - 1


def _gffn_body(te_ref, x_ref, w1_ref, b1_ref, w2_ref, b2_ref, out_ref):
    del te_ref
    x = x_ref[...]
    h = jnp.dot(x, w1_ref[0], preferred_element_type=jnp.float32) + b1_ref[0]
    a = h[:, :_DFF]
    g = h[:, _DFF:]
    act = a * (g / (1.0 + jnp.exp(-g)))
    out_ref[...] = jnp.dot(act, w2_ref[0],
                           preferred_element_type=jnp.float32) + b2_ref[0]


def kernel(x, router_W, router_temp, W1, b1, W2, b2):
    B, S, D = x.shape
    x_flat = x.reshape(-1, D)
    T = x_flat.shape[0]
    nt = (_K * T) // _TILE + _E
    rows = nt * _TILE

    pos0, pos1, w0, w1, te = pl.pallas_call(
        _router_body,
        out_shape=[
            jax.ShapeDtypeStruct((T, 1), jnp.int32),
            jax.ShapeDtypeStruct((T, 1), jnp.int32),
            jax.ShapeDtypeStruct((T, 1), jnp.float32),
            jax.ShapeDtypeStruct((T, 1), jnp.float32),
            jax.ShapeDtypeStruct((nt, 1), jnp.int32),
        ],
    )(x_flat, router_W, router_temp.reshape(1, 1))

    p0 = pos0.reshape(T)
    p1 = pos1.reshape(T)
    tile_expert = te.reshape(nt)

    x_sorted = jnp.zeros((rows, D), jnp.float32)
    x_sorted = x_sorted.at[p0].set(x_flat).at[p1].set(x_flat)

    y_sorted = pl.pallas_call(
        _gffn_body,
        grid_spec=pltpu.PrefetchScalarGridSpec(
            num_scalar_prefetch=1,
            grid=(nt,),
            in_specs=[
                pl.BlockSpec((_TILE, D), lambda i, te: (i, 0)),
                pl.BlockSpec((1, D, 2 * _DFF), lambda i, te: (te[i], 0, 0)),
                pl.BlockSpec((1, 1, 2 * _DFF), lambda i, te: (te[i], 0, 0)),
                pl.BlockSpec((1, _DFF, D), lambda i, te: (te[i], 0, 0)),
                pl.BlockSpec((1, 1, D), lambda i, te: (te[i], 0, 0)),
            ],
            out_specs=pl.BlockSpec((_TILE, D), lambda i, te: (i, 0)),
        ),
        out_shape=jax.ShapeDtypeStruct((rows, D), jnp.float32),
        compiler_params=pltpu.CompilerParams(
            dimension_semantics=("arbitrary",)),
    )(tile_expert, x_sorted, W1, b1.reshape(_E, 1, 2 * _DFF), W2,
      b2.reshape(_E, 1, D))

    out = y_sorted[p0] * w0 + y_sorted[p1] * w1
    return out.reshape(B, S, D)


# speedup vs baseline: 2.4141x; 1.1561x over previous
import functools
import jax
import jax.numpy as jnp
from jax import lax
from jax.experimental import pallas as pl
from jax.experimental.pallas import tpu as pltpu
from jax.experimental.pallas import tpu_sc as plsc

_NW = 32

_E = 8
_K = 2
_D = 768
_DFF = 2048
_TILE = 128


def _incl_cumsum_ax0(v):
    s = 1
    while s < v.shape[0]:
        v = v + jnp.concatenate(
            [jnp.zeros((s, v.shape[1]), v.dtype), v[:-s]], axis=0)
        s *= 2
    return v


def _router_body(x_ref, rw_ref, temp_ref, pos0_ref, pos1_ref, w0_ref, w1_ref,
                 te_ref):
    nt = te_ref.shape[0]
    x = x_ref[...]
    logits = jax.lax.dot_general(
        x, rw_ref[...], (((1,), (1,)), ((), ())),
        preferred_element_type=jnp.float32)
    logits = logits / temp_ref[...]
    t_rows = logits.shape[0]
    lane = jax.lax.broadcasted_iota(jnp.int32, (t_rows, _E), 1)
    m0 = jnp.max(logits, axis=1, keepdims=True)
    e0 = jnp.min(jnp.where(logits == m0, lane, _E), axis=1, keepdims=True)
    masked = jnp.where(lane == e0, -jnp.inf, logits)
    m1 = jnp.max(masked, axis=1, keepdims=True)
    e1 = jnp.min(jnp.where(masked == m1, lane, _E), axis=1, keepdims=True)
    w0_ref[...] = 1.0 / (1.0 + jnp.exp(m1 - m0))
    w1_ref[...] = 1.0 - w0_ref[...]

    oh0 = (lane == e0).astype(jnp.int32)
    oh1 = (lane == e1).astype(jnp.int32)
    c0 = _incl_cumsum_ax0(oh0)
    c1 = _incl_cumsum_ax0(oh1)
    total0 = c0[t_rows - 1:t_rows, :]
    total1 = c1[t_rows - 1:t_rows, :]
    counts = total0 + total1
    pt = (counts + (_TILE - 1)) // _TILE
    ltri = (jax.lax.broadcasted_iota(jnp.int32, (_E, _E), 0)
            < jax.lax.broadcasted_iota(jnp.int32, (_E, _E), 1))
    ts = jnp.dot(pt.astype(jnp.float32), ltri.astype(jnp.float32),
                 preferred_element_type=jnp.float32).astype(jnp.int32)
    ao = ts * _TILE
    rank0 = jnp.sum(oh0 * c0, axis=1, keepdims=True) - 1
    rank1 = (jnp.sum(oh1 * c1, axis=1, keepdims=True) - 1
             + jnp.sum(oh1 * total0, axis=1, keepdims=True))
    pos0_ref[...] = jnp.sum(oh0 * ao, axis=1, keepdims=True) + rank0
    pos1_ref[...] = jnp.sum(oh1 * ao, axis=1, keepdims=True) + rank1
    rowi = jax.lax.broadcasted_iota(jnp.int32, (nt, _E), 0)
    te_ref[...] = jnp.sum((rowi >= ts).astype(jnp.int32), axis=1,
                          keepdims=True) - 1


def _make_dispatch(T, D, rows):
    per_w = T // _NW
    ch = 32
    n_g = per_w // ch
    mesh = plsc.VectorSubcoreMesh(core_axis_name="c", subcore_axis_name="s")

    @functools.partial(
        pl.kernel, mesh=mesh,
        out_type=jax.ShapeDtypeStruct((rows, D), jnp.float32),
        scratch_types=[
            pltpu.VMEM((ch,), jnp.int32),
            pltpu.VMEM((ch, D), jnp.float32),
            pltpu.SemaphoreType.DMA,
        ])
    def _dispatch(x_hbm, p0_hbm, p1_hbm, xs_hbm, idx_v, rows_v, sem):
        wid = lax.axis_index("s") * 2 + lax.axis_index("c")
        for g in range(n_g):
            base = wid * per_w + g * ch
            pltpu.sync_copy(x_hbm.at[pl.ds(base, ch)], rows_v)
            pltpu.sync_copy(p0_hbm.at[pl.ds(base, ch)], idx_v)
            pltpu.async_copy(rows_v, xs_hbm.at[idx_v], sem).wait()
            pltpu.sync_copy(p1_hbm.at[pl.ds(base, ch)], idx_v)
            pltpu.async_copy(rows_v, xs_hbm.at[idx_v], sem).wait()

    return _dispatch


def _make_cgather(T, D, rows):
    per_w = T // _NW
    ch = 32
    n_g = per_w // ch
    mesh = plsc.VectorSubcoreMesh(core_axis_name="c", subcore_axis_name="s")

    @functools.partial(
        pl.kernel, mesh=mesh,
        out_type=[
            jax.ShapeDtypeStruct((T, D), jnp.float32),
            jax.ShapeDtypeStruct((T, D), jnp.float32),
        ],
        scratch_types=[
            pltpu.VMEM((ch,), jnp.int32),
            pltpu.VMEM((ch,), jnp.int32),
            pltpu.VMEM((ch, D), jnp.float32),
            pltpu.VMEM((ch, D), jnp.float32),
            pltpu.SemaphoreType.DMA,
        ])
    def _cgather(ys_hbm, p0_hbm, p1_hbm, y0_hbm, y1_hbm,
                 i0_v, i1_v, r0_v, r1_v, sem):
        wid = lax.axis_index("s") * 2 + lax.axis_index("c")
        for g in range(n_g):
            base = wid * per_w + g * ch
            pltpu.sync_copy(p0_hbm.at[pl.ds(base, ch)], i0_v)
            pltpu.sync_copy(p1_hbm.at[pl.ds(base, ch)], i1_v)
            pltpu.async_copy(ys_hbm.at[i0_v], r0_v, sem).wait()
            pltpu.sync_copy(r0_v, y0_hbm.at[pl.ds(base, ch)])
            pltpu.async_copy(ys_hbm.at[i1_v], r1_v, sem).wait()
            pltpu.sync_copy(r1_v, y1_hbm.at[pl.ds(base, ch)])

    return _cgather


def _wadd_body(y0_ref, y1_ref, w0_ref, w1_ref, out_ref):
    out_ref[...] = (y0_ref[...] * w0_ref[...] + y1_ref[...] * w1_ref[...])


def _gffn_body(te_ref, x_ref, w1_ref, b1_ref, w2_ref, b2_ref, out_ref):
    del te_ref
    x = x_ref[...]
    h = jnp.dot(x, w1_ref[0], preferred_element_type=jnp.float32) + b1_ref[0]
    a = h[:, :_DFF]
    g = h[:, _DFF:]
    act = a * (g / (1.0 + jnp.exp(-g)))
    out_ref[...] = jnp.dot(act, w2_ref[0],
                           preferred_element_type=jnp.float32) + b2_ref[0]


def kernel(x, router_W, router_temp, W1, b1, W2, b2):
    B, S, D = x.shape
    x_flat = x.reshape(-1, D)
    T = x_flat.shape[0]
    nt = (_K * T) // _TILE + _E
    rows = nt * _TILE

    pos0, pos1, w0, w1, te = pl.pallas_call(
        _router_body,
        out_shape=[
            jax.ShapeDtypeStruct((T, 1), jnp.int32),
            jax.ShapeDtypeStruct((T, 1), jnp.int32),
            jax.ShapeDtypeStruct((T, 1), jnp.float32),
            jax.ShapeDtypeStruct((T, 1), jnp.float32),
            jax.ShapeDtypeStruct((nt, 1), jnp.int32),
        ],
    )(x_flat, router_W, router_temp.reshape(1, 1))

    p0 = pos0.reshape(T)
    p1 = pos1.reshape(T)
    tile_expert = te.reshape(nt)

    x_sorted = _make_dispatch(T, D, rows)(x_flat, p0, p1)

    y_sorted = pl.pallas_call(
        _gffn_body,
        grid_spec=pltpu.PrefetchScalarGridSpec(
            num_scalar_prefetch=1,
            grid=(nt,),
            in_specs=[
                pl.BlockSpec((_TILE, D), lambda i, te: (i, 0)),
                pl.BlockSpec((1, D, 2 * _DFF), lambda i, te: (te[i], 0, 0)),
                pl.BlockSpec((1, 1, 2 * _DFF), lambda i, te: (te[i], 0, 0)),
                pl.BlockSpec((1, _DFF, D), lambda i, te: (te[i], 0, 0)),
                pl.BlockSpec((1, 1, D), lambda i, te: (te[i], 0, 0)),
            ],
            out_specs=pl.BlockSpec((_TILE, D), lambda i, te: (i, 0)),
        ),
        out_shape=jax.ShapeDtypeStruct((rows, D), jnp.float32),
        compiler_params=pltpu.CompilerParams(
            dimension_semantics=("arbitrary",)),
    )(tile_expert, x_sorted, W1, b1.reshape(_E, 1, 2 * _DFF), W2,
      b2.reshape(_E, 1, D))

    y0, y1 = _make_cgather(T, D, rows)(y_sorted, p0, p1)
    tt = 1024
    out = pl.pallas_call(
        _wadd_body,
        grid=(T // tt,),
        in_specs=[
            pl.BlockSpec((tt, D), lambda i: (i, 0)),
            pl.BlockSpec((tt, D), lambda i: (i, 0)),
            pl.BlockSpec((tt, 1), lambda i: (i, 0)),
            pl.BlockSpec((tt, 1), lambda i: (i, 0)),
        ],
        out_specs=pl.BlockSpec((tt, D), lambda i: (i, 0)),
        out_shape=jax.ShapeDtypeStruct((T, D), jnp.float32),
    )(y0, y1, w0, w1)
    return out.reshape(B, S, D)


# speedup vs baseline: 2.6148x; 1.0832x over previous
import functools
import jax
import jax.numpy as jnp
from jax import lax
from jax.experimental import pallas as pl
from jax.experimental.pallas import tpu as pltpu
from jax.experimental.pallas import tpu_sc as plsc

_NW = 32

_E = 8
_K = 2
_D = 768
_DFF = 2048
_TILE = 256


def _incl_cumsum_ax0(v):
    s = 1
    while s < v.shape[0]:
        v = v + jnp.concatenate(
            [jnp.zeros((s, v.shape[1]), v.dtype), v[:-s]], axis=0)
        s *= 2
    return v


def _router_body(x_ref, rw_ref, temp_ref, pos0_ref, pos1_ref, w0_ref, w1_ref,
                 te_ref):
    nt = te_ref.shape[0]
    x = x_ref[...]
    logits = jax.lax.dot_general(
        x, rw_ref[...], (((1,), (1,)), ((), ())),
        preferred_element_type=jnp.float32)
    logits = logits / temp_ref[...]
    t_rows = logits.shape[0]
    lane = jax.lax.broadcasted_iota(jnp.int32, (t_rows, _E), 1)
    m0 = jnp.max(logits, axis=1, keepdims=True)
    e0 = jnp.min(jnp.where(logits == m0, lane, _E), axis=1, keepdims=True)
    masked = jnp.where(lane == e0, -jnp.inf, logits)
    m1 = jnp.max(masked, axis=1, keepdims=True)
    e1 = jnp.min(jnp.where(masked == m1, lane, _E), axis=1, keepdims=True)
    w0_ref[...] = 1.0 / (1.0 + jnp.exp(m1 - m0))
    w1_ref[...] = 1.0 - w0_ref[...]

    oh0 = (lane == e0).astype(jnp.int32)
    oh1 = (lane == e1).astype(jnp.int32)
    c0 = _incl_cumsum_ax0(oh0)
    c1 = _incl_cumsum_ax0(oh1)
    total0 = c0[t_rows - 1:t_rows, :]
    total1 = c1[t_rows - 1:t_rows, :]
    counts = total0 + total1
    pt = (counts + (_TILE - 1)) // _TILE
    ltri = (jax.lax.broadcasted_iota(jnp.int32, (_E, _E), 0)
            < jax.lax.broadcasted_iota(jnp.int32, (_E, _E), 1))
    ts = jnp.dot(pt.astype(jnp.float32), ltri.astype(jnp.float32),
                 preferred_element_type=jnp.float32).astype(jnp.int32)
    ao = ts * _TILE
    rank0 = jnp.sum(oh0 * c0, axis=1, keepdims=True) - 1
    rank1 = (jnp.sum(oh1 * c1, axis=1, keepdims=True) - 1
             + jnp.sum(oh1 * total0, axis=1, keepdims=True))
    pos0_ref[...] = jnp.sum(oh0 * ao, axis=1, keepdims=True) + rank0
    pos1_ref[...] = jnp.sum(oh1 * ao, axis=1, keepdims=True) + rank1
    rowi = jax.lax.broadcasted_iota(jnp.int32, (nt, _E), 0)
    te_ref[...] = jnp.sum((rowi >= ts).astype(jnp.int32), axis=1,
                          keepdims=True) - 1


def _make_dispatch(T, D, rows):
    per_w = T // _NW
    ch = 32
    n_g = per_w // ch
    mesh = plsc.VectorSubcoreMesh(core_axis_name="c", subcore_axis_name="s")

    @functools.partial(
        pl.kernel, mesh=mesh,
        out_type=jax.ShapeDtypeStruct((rows, D), jnp.float32),
        scratch_types=[
            pltpu.VMEM((ch,), jnp.int32),
            pltpu.VMEM((ch, D), jnp.float32),
            pltpu.SemaphoreType.DMA,
        ])
    def _dispatch(x_hbm, p0_hbm, p1_hbm, xs_hbm, idx_v, rows_v, sem):
        wid = lax.axis_index("s") * 2 + lax.axis_index("c")
        for g in range(n_g):
            base = wid * per_w + g * ch
            pltpu.sync_copy(x_hbm.at[pl.ds(base, ch)], rows_v)
            pltpu.sync_copy(p0_hbm.at[pl.ds(base, ch)], idx_v)
            pltpu.async_copy(rows_v, xs_hbm.at[idx_v], sem).wait()
            pltpu.sync_copy(p1_hbm.at[pl.ds(base, ch)], idx_v)
            pltpu.async_copy(rows_v, xs_hbm.at[idx_v], sem).wait()

    return _dispatch


def _make_cgather(T, D, rows):
    per_w = T // _NW
    ch = 32
    n_g = per_w // ch
    mesh = plsc.VectorSubcoreMesh(core_axis_name="c", subcore_axis_name="s")

    @functools.partial(
        pl.kernel, mesh=mesh,
        out_type=[
            jax.ShapeDtypeStruct((T, D), jnp.float32),
            jax.ShapeDtypeStruct((T, D), jnp.float32),
        ],
        scratch_types=[
            pltpu.VMEM((ch,), jnp.int32),
            pltpu.VMEM((ch,), jnp.int32),
            pltpu.VMEM((ch, D), jnp.float32),
            pltpu.VMEM((ch, D), jnp.float32),
            pltpu.SemaphoreType.DMA,
        ])
    def _cgather(ys_hbm, p0_hbm, p1_hbm, y0_hbm, y1_hbm,
                 i0_v, i1_v, r0_v, r1_v, sem):
        wid = lax.axis_index("s") * 2 + lax.axis_index("c")
        for g in range(n_g):
            base = wid * per_w + g * ch
            pltpu.sync_copy(p0_hbm.at[pl.ds(base, ch)], i0_v)
            pltpu.sync_copy(p1_hbm.at[pl.ds(base, ch)], i1_v)
            pltpu.async_copy(ys_hbm.at[i0_v], r0_v, sem).wait()
            pltpu.sync_copy(r0_v, y0_hbm.at[pl.ds(base, ch)])
            pltpu.async_copy(ys_hbm.at[i1_v], r1_v, sem).wait()
            pltpu.sync_copy(r1_v, y1_hbm.at[pl.ds(base, ch)])

    return _cgather


def _wadd_body(y0_ref, y1_ref, w0_ref, w1_ref, out_ref):
    out_ref[...] = (y0_ref[...] * w0_ref[...] + y1_ref[...] * w1_ref[...])


def _gffn_body(te_ref, x_ref, w1_ref, b1_ref, w2_ref, b2_ref, out_ref):
    del te_ref
    x = x_ref[...]
    h = jnp.dot(x, w1_ref[0], preferred_element_type=jnp.float32) + b1_ref[0]
    a = h[:, :_DFF]
    g = h[:, _DFF:]
    act = a * (g / (1.0 + jnp.exp(-g)))
    out_ref[...] = jnp.dot(act, w2_ref[0],
                           preferred_element_type=jnp.float32) + b2_ref[0]


def kernel(x, router_W, router_temp, W1, b1, W2, b2):
    B, S, D = x.shape
    x_flat = x.reshape(-1, D)
    T = x_flat.shape[0]
    nt = (_K * T) // _TILE + _E
    rows = nt * _TILE

    pos0, pos1, w0, w1, te = pl.pallas_call(
        _router_body,
        out_shape=[
            jax.ShapeDtypeStruct((T, 1), jnp.int32),
            jax.ShapeDtypeStruct((T, 1), jnp.int32),
            jax.ShapeDtypeStruct((T, 1), jnp.float32),
            jax.ShapeDtypeStruct((T, 1), jnp.float32),
            jax.ShapeDtypeStruct((nt, 1), jnp.int32),
        ],
    )(x_flat, router_W, router_temp.reshape(1, 1))

    p0 = pos0.reshape(T)
    p1 = pos1.reshape(T)
    tile_expert = te.reshape(nt)

    x_sorted = _make_dispatch(T, D, rows)(x_flat, p0, p1)

    y_sorted = pl.pallas_call(
        _gffn_body,
        grid_spec=pltpu.PrefetchScalarGridSpec(
            num_scalar_prefetch=1,
            grid=(nt,),
            in_specs=[
                pl.BlockSpec((_TILE, D), lambda i, te: (i, 0)),
                pl.BlockSpec((1, D, 2 * _DFF), lambda i, te: (te[i], 0, 0)),
                pl.BlockSpec((1, 1, 2 * _DFF), lambda i, te: (te[i], 0, 0)),
                pl.BlockSpec((1, _DFF, D), lambda i, te: (te[i], 0, 0)),
                pl.BlockSpec((1, 1, D), lambda i, te: (te[i], 0, 0)),
            ],
            out_specs=pl.BlockSpec((_TILE, D), lambda i, te: (i, 0)),
        ),
        out_shape=jax.ShapeDtypeStruct((rows, D), jnp.float32),
        compiler_params=pltpu.CompilerParams(
            dimension_semantics=("arbitrary",)),
    )(tile_expert, x_sorted, W1, b1.reshape(_E, 1, 2 * _DFF), W2,
      b2.reshape(_E, 1, D))

    y0, y1 = _make_cgather(T, D, rows)(y_sorted, p0, p1)
    tt = 1024
    out = pl.pallas_call(
        _wadd_body,
        grid=(T // tt,),
        in_specs=[
            pl.BlockSpec((tt, D), lambda i: (i, 0)),
            pl.BlockSpec((tt, D), lambda i: (i, 0)),
            pl.BlockSpec((tt, 1), lambda i: (i, 0)),
            pl.BlockSpec((tt, 1), lambda i: (i, 0)),
        ],
        out_specs=pl.BlockSpec((tt, D), lambda i: (i, 0)),
        out_shape=jax.ShapeDtypeStruct((T, D), jnp.float32),
    )(y0, y1, w0, w1)
    return out.reshape(B, S, D)


# speedup vs baseline: 2.6222x; 1.0028x over previous
import functools
import jax
import jax.numpy as jnp
from jax import lax
from jax.experimental import pallas as pl
from jax.experimental.pallas import tpu as pltpu
from jax.experimental.pallas import tpu_sc as plsc

_NW = 32

_E = 8
_K = 2
_D = 768
_DFF = 2048
_TILE = 256


def _incl_cumsum_ax0(v):
    s = 1
    while s < v.shape[0]:
        v = v + jnp.concatenate(
            [jnp.zeros((s, v.shape[1]), v.dtype), v[:-s]], axis=0)
        s *= 2
    return v


def _router_body(x_ref, rw_ref, temp_ref, pos0_ref, pos1_ref, w0_ref, w1_ref,
                 te_ref):
    nt = te_ref.shape[0]
    x = x_ref[...]
    logits = jax.lax.dot_general(
        x, rw_ref[...], (((1,), (1,)), ((), ())),
        preferred_element_type=jnp.float32)
    logits = logits / temp_ref[...]
    t_rows = logits.shape[0]
    lane = jax.lax.broadcasted_iota(jnp.int32, (t_rows, _E), 1)
    m0 = jnp.max(logits, axis=1, keepdims=True)
    e0 = jnp.min(jnp.where(logits == m0, lane, _E), axis=1, keepdims=True)
    masked = jnp.where(lane == e0, -jnp.inf, logits)
    m1 = jnp.max(masked, axis=1, keepdims=True)
    e1 = jnp.min(jnp.where(masked == m1, lane, _E), axis=1, keepdims=True)
    w0_ref[...] = 1.0 / (1.0 + jnp.exp(m1 - m0))
    w1_ref[...] = 1.0 - w0_ref[...]

    oh0 = (lane == e0).astype(jnp.int32)
    oh1 = (lane == e1).astype(jnp.int32)
    c0 = _incl_cumsum_ax0(oh0)
    c1 = _incl_cumsum_ax0(oh1)
    total0 = c0[t_rows - 1:t_rows, :]
    total1 = c1[t_rows - 1:t_rows, :]
    counts = total0 + total1
    pt = (counts + (_TILE - 1)) // _TILE
    ltri = (jax.lax.broadcasted_iota(jnp.int32, (_E, _E), 0)
            < jax.lax.broadcasted_iota(jnp.int32, (_E, _E), 1))
    ts = jnp.dot(pt.astype(jnp.float32), ltri.astype(jnp.float32),
                 preferred_element_type=jnp.float32).astype(jnp.int32)
    ao = ts * _TILE
    rank0 = jnp.sum(oh0 * c0, axis=1, keepdims=True) - 1
    rank1 = (jnp.sum(oh1 * c1, axis=1, keepdims=True) - 1
             + jnp.sum(oh1 * total0, axis=1, keepdims=True))
    pos0_ref[...] = jnp.sum(oh0 * ao, axis=1, keepdims=True) + rank0
    pos1_ref[...] = jnp.sum(oh1 * ao, axis=1, keepdims=True) + rank1
    rowi = jax.lax.broadcasted_iota(jnp.int32, (nt, _E), 0)
    te_ref[...] = jnp.sum((rowi >= ts).astype(jnp.int32), axis=1,
                          keepdims=True) - 1


def _make_dispatch(T, D, rows):
    per_w = T // _NW
    ch = 32
    n_g = per_w // ch
    mesh = plsc.VectorSubcoreMesh(core_axis_name="c", subcore_axis_name="s")

    @functools.partial(
        pl.kernel, mesh=mesh,
        out_type=jax.ShapeDtypeStruct((rows, D), jnp.float32),
        scratch_types=[
            pltpu.VMEM((ch,), jnp.int32),
            pltpu.VMEM((ch, D), jnp.float32),
            pltpu.SemaphoreType.DMA,
        ])
    def _dispatch(x_hbm, p0_hbm, p1_hbm, xs_hbm, idx_v, rows_v, sem):
        wid = lax.axis_index("s") * 2 + lax.axis_index("c")
        for g in range(n_g):
            base = wid * per_w + g * ch
            pltpu.sync_copy(x_hbm.at[pl.ds(base, ch)], rows_v)
            pltpu.sync_copy(p0_hbm.at[pl.ds(base, ch)], idx_v)
            pltpu.async_copy(rows_v, xs_hbm.at[idx_v], sem).wait()
            pltpu.sync_copy(p1_hbm.at[pl.ds(base, ch)], idx_v)
            pltpu.async_copy(rows_v, xs_hbm.at[idx_v], sem).wait()

    return _dispatch


def _make_cgather(T, D, rows):
    per_w = T // _NW
    ch = 32
    n_g = per_w // ch
    mesh = plsc.VectorSubcoreMesh(core_axis_name="c", subcore_axis_name="s")

    @functools.partial(
        pl.kernel, mesh=mesh,
        out_type=[
            jax.ShapeDtypeStruct((T, D), jnp.float32),
            jax.ShapeDtypeStruct((T, D), jnp.float32),
        ],
        scratch_types=[
            pltpu.VMEM((ch,), jnp.int32),
            pltpu.VMEM((ch,), jnp.int32),
            pltpu.VMEM((ch, D), jnp.float32),
            pltpu.VMEM((ch, D), jnp.float32),
            pltpu.SemaphoreType.DMA,
        ])
    def _cgather(ys_hbm, p0_hbm, p1_hbm, y0_hbm, y1_hbm,
                 i0_v, i1_v, r0_v, r1_v, sem):
        wid = lax.axis_index("s") * 2 + lax.axis_index("c")
        for g in range(n_g):
            base = wid * per_w + g * ch
            pltpu.sync_copy(p0_hbm.at[pl.ds(base, ch)], i0_v)
            pltpu.sync_copy(p1_hbm.at[pl.ds(base, ch)], i1_v)
            pltpu.async_copy(ys_hbm.at[i0_v], r0_v, sem).wait()
            pltpu.sync_copy(r0_v, y0_hbm.at[pl.ds(base, ch)])
            pltpu.async_copy(ys_hbm.at[i1_v], r1_v, sem).wait()
            pltpu.sync_copy(r1_v, y1_hbm.at[pl.ds(base, ch)])

    return _cgather


def _wadd_body(y0_ref, y1_ref, w0_ref, w1_ref, out_ref):
    out_ref[...] = (y0_ref[...] * w0_ref[...] + y1_ref[...] * w1_ref[...])


def _gffn_body(te_ref, x_ref, w1_ref, b1_ref, w2_ref, b2_ref, out_ref):
    del te_ref
    x = x_ref[...].astype(jnp.bfloat16)
    h = jnp.dot(x, w1_ref[0].astype(jnp.bfloat16),
                preferred_element_type=jnp.float32) + b1_ref[0]
    a = h[:, :_DFF]
    g = h[:, _DFF:]
    act = a * (g / (1.0 + jnp.exp(-g)))
    out_ref[...] = jnp.dot(act.astype(jnp.bfloat16),
                           w2_ref[0].astype(jnp.bfloat16),
                           preferred_element_type=jnp.float32) + b2_ref[0]


def kernel(x, router_W, router_temp, W1, b1, W2, b2):
    B, S, D = x.shape
    x_flat = x.reshape(-1, D)
    T = x_flat.shape[0]
    nt = (_K * T) // _TILE + _E
    rows = nt * _TILE

    pos0, pos1, w0, w1, te = pl.pallas_call(
        _router_body,
        out_shape=[
            jax.ShapeDtypeStruct((T, 1), jnp.int32),
            jax.ShapeDtypeStruct((T, 1), jnp.int32),
            jax.ShapeDtypeStruct((T, 1), jnp.float32),
            jax.ShapeDtypeStruct((T, 1), jnp.float32),
            jax.ShapeDtypeStruct((nt, 1), jnp.int32),
        ],
    )(x_flat, router_W, router_temp.reshape(1, 1))

    p0 = pos0.reshape(T)
    p1 = pos1.reshape(T)
    tile_expert = te.reshape(nt)

    x_sorted = _make_dispatch(T, D, rows)(x_flat, p0, p1)

    y_sorted = pl.pallas_call(
        _gffn_body,
        grid_spec=pltpu.PrefetchScalarGridSpec(
            num_scalar_prefetch=1,
            grid=(nt,),
            in_specs=[
                pl.BlockSpec((_TILE, D), lambda i, te: (i, 0)),
                pl.BlockSpec((1, D, 2 * _DFF), lambda i, te: (te[i], 0, 0)),
                pl.BlockSpec((1, 1, 2 * _DFF), lambda i, te: (te[i], 0, 0)),
                pl.BlockSpec((1, _DFF, D), lambda i, te: (te[i], 0, 0)),
                pl.BlockSpec((1, 1, D), lambda i, te: (te[i], 0, 0)),
            ],
            out_specs=pl.BlockSpec((_TILE, D), lambda i, te: (i, 0)),
        ),
        out_shape=jax.ShapeDtypeStruct((rows, D), jnp.float32),
        compiler_params=pltpu.CompilerParams(
            dimension_semantics=("arbitrary",)),
    )(tile_expert, x_sorted, W1, b1.reshape(_E, 1, 2 * _DFF), W2,
      b2.reshape(_E, 1, D))

    y0, y1 = _make_cgather(T, D, rows)(y_sorted, p0, p1)
    tt = 1024
    out = pl.pallas_call(
        _wadd_body,
        grid=(T // tt,),
        in_specs=[
            pl.BlockSpec((tt, D), lambda i: (i, 0)),
            pl.BlockSpec((tt, D), lambda i: (i, 0)),
            pl.BlockSpec((tt, 1), lambda i: (i, 0)),
            pl.BlockSpec((tt, 1), lambda i: (i, 0)),
        ],
        out_specs=pl.BlockSpec((tt, D), lambda i: (i, 0)),
        out_shape=jax.ShapeDtypeStruct((T, D), jnp.float32),
    )(y0, y1, w0, w1)
    return out.reshape(B, S, D)


# speedup vs baseline: 2.6947x; 1.0276x over previous
import functools
import jax
import jax.numpy as jnp
from jax import lax
from jax.experimental import pallas as pl
from jax.experimental.pallas import tpu as pltpu
from jax.experimental.pallas import tpu_sc as plsc

_NW = 32

_E = 8
_K = 2
_D = 768
_DFF = 2048
_TILE = 256


def _incl_cumsum_ax0(v):
    s = 1
    while s < v.shape[0]:
        v = v + jnp.concatenate(
            [jnp.zeros((s, v.shape[1]), v.dtype), v[:-s]], axis=0)
        s *= 2
    return v


def _router_body(x_ref, rw_ref, temp_ref, pos0_ref, pos1_ref, w0_ref, w1_ref,
                 te_ref, nu_ref):
    nt = te_ref.shape[0]
    x = x_ref[...]
    logits = jax.lax.dot_general(
        x, rw_ref[...], (((1,), (1,)), ((), ())),
        preferred_element_type=jnp.float32)
    logits = logits / temp_ref[...]
    t_rows = logits.shape[0]
    lane = jax.lax.broadcasted_iota(jnp.int32, (t_rows, _E), 1)
    m0 = jnp.max(logits, axis=1, keepdims=True)
    e0 = jnp.min(jnp.where(logits == m0, lane, _E), axis=1, keepdims=True)
    masked = jnp.where(lane == e0, -jnp.inf, logits)
    m1 = jnp.max(masked, axis=1, keepdims=True)
    e1 = jnp.min(jnp.where(masked == m1, lane, _E), axis=1, keepdims=True)
    w0_ref[...] = 1.0 / (1.0 + jnp.exp(m1 - m0))
    w1_ref[...] = 1.0 - w0_ref[...]

    oh0 = (lane == e0).astype(jnp.int32)
    oh1 = (lane == e1).astype(jnp.int32)
    c0 = _incl_cumsum_ax0(oh0)
    c1 = _incl_cumsum_ax0(oh1)
    total0 = c0[t_rows - 1:t_rows, :]
    total1 = c1[t_rows - 1:t_rows, :]
    counts = total0 + total1
    pt = (counts + (_TILE - 1)) // _TILE
    ltri = (jax.lax.broadcasted_iota(jnp.int32, (_E, _E), 0)
            < jax.lax.broadcasted_iota(jnp.int32, (_E, _E), 1))
    ts = jnp.dot(pt.astype(jnp.float32), ltri.astype(jnp.float32),
                 preferred_element_type=jnp.float32).astype(jnp.int32)
    ao = ts * _TILE
    rank0 = jnp.sum(oh0 * c0, axis=1, keepdims=True) - 1
    rank1 = (jnp.sum(oh1 * c1, axis=1, keepdims=True) - 1
             + jnp.sum(oh1 * total0, axis=1, keepdims=True))
    pos0_ref[...] = jnp.sum(oh0 * ao, axis=1, keepdims=True) + rank0
    pos1_ref[...] = jnp.sum(oh1 * ao, axis=1, keepdims=True) + rank1
    rowi = jax.lax.broadcasted_iota(jnp.int32, (nt, _E), 0)
    te_ref[...] = jnp.sum((rowi >= ts).astype(jnp.int32), axis=1,
                          keepdims=True) - 1
    nu_ref[...] = jnp.sum(pt, axis=1, keepdims=True)


def _make_dispatch(T, D, rows):
    per_w = T // _NW
    ch = 32
    n_g = per_w // ch
    mesh = plsc.VectorSubcoreMesh(core_axis_name="c", subcore_axis_name="s")

    @functools.partial(
        pl.kernel, mesh=mesh,
        out_type=jax.ShapeDtypeStruct((rows, D), jnp.float32),
        scratch_types=[
            pltpu.VMEM((ch,), jnp.int32),
            pltpu.VMEM((ch, D), jnp.float32),
            pltpu.SemaphoreType.DMA,
        ])
    def _dispatch(x_hbm, p0_hbm, p1_hbm, xs_hbm, idx_v, rows_v, sem):
        wid = lax.axis_index("s") * 2 + lax.axis_index("c")
        for g in range(n_g):
            base = wid * per_w + g * ch
            pltpu.sync_copy(x_hbm.at[pl.ds(base, ch)], rows_v)
            pltpu.sync_copy(p0_hbm.at[pl.ds(base, ch)], idx_v)
            pltpu.async_copy(rows_v, xs_hbm.at[idx_v], sem).wait()
            pltpu.sync_copy(p1_hbm.at[pl.ds(base, ch)], idx_v)
            pltpu.async_copy(rows_v, xs_hbm.at[idx_v], sem).wait()

    return _dispatch


def _make_cgather(T, D, rows):
    per_w = T // _NW
    ch = 32
    n_g = per_w // ch
    mesh = plsc.VectorSubcoreMesh(core_axis_name="c", subcore_axis_name="s")

    @functools.partial(
        pl.kernel, mesh=mesh,
        out_type=[
            jax.ShapeDtypeStruct((T, D), jnp.float32),
            jax.ShapeDtypeStruct((T, D), jnp.float32),
        ],
        scratch_types=[
            pltpu.VMEM((ch,), jnp.int32),
            pltpu.VMEM((ch,), jnp.int32),
            pltpu.VMEM((ch, D), jnp.float32),
            pltpu.VMEM((ch, D), jnp.float32),
            pltpu.SemaphoreType.DMA,
        ])
    def _cgather(ys_hbm, p0_hbm, p1_hbm, y0_hbm, y1_hbm,
                 i0_v, i1_v, r0_v, r1_v, sem):
        wid = lax.axis_index("s") * 2 + lax.axis_index("c")
        for g in range(n_g):
            base = wid * per_w + g * ch
            pltpu.sync_copy(p0_hbm.at[pl.ds(base, ch)], i0_v)
            pltpu.sync_copy(p1_hbm.at[pl.ds(base, ch)], i1_v)
            pltpu.async_copy(ys_hbm.at[i0_v], r0_v, sem).wait()
            pltpu.sync_copy(r0_v, y0_hbm.at[pl.ds(base, ch)])
            pltpu.async_copy(ys_hbm.at[i1_v], r1_v, sem).wait()
            pltpu.sync_copy(r1_v, y1_hbm.at[pl.ds(base, ch)])

    return _cgather


def _wadd_body(y0_ref, y1_ref, w0_ref, w1_ref, out_ref):
    out_ref[...] = (y0_ref[...] * w0_ref[...] + y1_ref[...] * w1_ref[...])


def _gffn_body(te_ref, nu_ref, x_ref, w1_ref, b1_ref, w2_ref, b2_ref,
               out_ref):
    del te_ref
    i = pl.program_id(0)

    @pl.when(i < nu_ref[0])
    def _():
        x = x_ref[...].astype(jnp.bfloat16)
        h = jnp.dot(x, w1_ref[0].astype(jnp.bfloat16),
                    preferred_element_type=jnp.float32) + b1_ref[0]
        a = h[:, :_DFF]
        g = h[:, _DFF:]
        act = a * (g / (1.0 + jnp.exp(-g)))
        out_ref[...] = jnp.dot(act.astype(jnp.bfloat16),
                               w2_ref[0].astype(jnp.bfloat16),
                               preferred_element_type=jnp.float32) + b2_ref[0]


def kernel(x, router_W, router_temp, W1, b1, W2, b2):
    B, S, D = x.shape
    x_flat = x.reshape(-1, D)
    T = x_flat.shape[0]
    nt = (_K * T) // _TILE + _E
    rows = nt * _TILE

    pos0, pos1, w0, w1, te, nu = pl.pallas_call(
        _router_body,
        out_shape=[
            jax.ShapeDtypeStruct((T, 1), jnp.int32),
            jax.ShapeDtypeStruct((T, 1), jnp.int32),
            jax.ShapeDtypeStruct((T, 1), jnp.float32),
            jax.ShapeDtypeStruct((T, 1), jnp.float32),
            jax.ShapeDtypeStruct((nt, 1), jnp.int32),
            jax.ShapeDtypeStruct((1, 1), jnp.int32),
        ],
    )(x_flat, router_W, router_temp.reshape(1, 1))

    p0 = pos0.reshape(T)
    p1 = pos1.reshape(T)
    tile_expert = te.reshape(nt)

    x_sorted = _make_dispatch(T, D, rows)(x_flat, p0, p1)

    y_sorted = pl.pallas_call(
        _gffn_body,
        grid_spec=pltpu.PrefetchScalarGridSpec(
            num_scalar_prefetch=2,
            grid=(nt,),
            in_specs=[
                pl.BlockSpec((_TILE, D), lambda i, te, nu: (i, 0)),
                pl.BlockSpec((1, D, 2 * _DFF),
                             lambda i, te, nu: (te[i], 0, 0)),
                pl.BlockSpec((1, 1, 2 * _DFF),
                             lambda i, te, nu: (te[i], 0, 0)),
                pl.BlockSpec((1, _DFF, D), lambda i, te, nu: (te[i], 0, 0)),
                pl.BlockSpec((1, 1, D), lambda i, te, nu: (te[i], 0, 0)),
            ],
            out_specs=pl.BlockSpec((_TILE, D), lambda i, te, nu: (i, 0)),
        ),
        out_shape=jax.ShapeDtypeStruct((rows, D), jnp.float32),
        compiler_params=pltpu.CompilerParams(
            dimension_semantics=("arbitrary",)),
    )(tile_expert, nu.reshape(1), x_sorted, W1,
      b1.reshape(_E, 1, 2 * _DFF), W2, b2.reshape(_E, 1, D))

    y0, y1 = _make_cgather(T, D, rows)(y_sorted, p0, p1)
    tt = 1024
    out = pl.pallas_call(
        _wadd_body,
        grid=(T // tt,),
        in_specs=[
            pl.BlockSpec((tt, D), lambda i: (i, 0)),
            pl.BlockSpec((tt, D), lambda i: (i, 0)),
            pl.BlockSpec((tt, 1), lambda i: (i, 0)),
            pl.BlockSpec((tt, 1), lambda i: (i, 0)),
        ],
        out_specs=pl.BlockSpec((tt, D), lambda i: (i, 0)),
        out_shape=jax.ShapeDtypeStruct((T, D), jnp.float32),
    )(y0, y1, w0, w1)
    return out.reshape(B, S, D)


# speedup vs baseline: 2.7905x; 1.0355x over previous
import functools
import jax
import jax.numpy as jnp
from jax import lax
from jax.experimental import pallas as pl
from jax.experimental.pallas import tpu as pltpu
from jax.experimental.pallas import tpu_sc as plsc

_NW = 32

_E = 8
_K = 2
_D = 768
_DFF = 2048
_TILE = 256


def _incl_cumsum_ax0(v):
    s = 1
    while s < v.shape[0]:
        v = v + jnp.concatenate(
            [jnp.zeros((s, v.shape[1]), v.dtype), v[:-s]], axis=0)
        s *= 2
    return v


def _router_body(x_ref, rw_ref, temp_ref, pos0_ref, pos1_ref, w0_ref, w1_ref,
                 te_ref, nu_ref):
    nt = te_ref.shape[0]
    x = x_ref[...]
    logits = jax.lax.dot_general(
        x, rw_ref[...], (((1,), (1,)), ((), ())),
        preferred_element_type=jnp.float32)
    logits = logits / temp_ref[...]
    t_rows = logits.shape[0]
    lane = jax.lax.broadcasted_iota(jnp.int32, (t_rows, _E), 1)
    m0 = jnp.max(logits, axis=1, keepdims=True)
    e0 = jnp.min(jnp.where(logits == m0, lane, _E), axis=1, keepdims=True)
    masked = jnp.where(lane == e0, -jnp.inf, logits)
    m1 = jnp.max(masked, axis=1, keepdims=True)
    e1 = jnp.min(jnp.where(masked == m1, lane, _E), axis=1, keepdims=True)
    w0_ref[...] = 1.0 / (1.0 + jnp.exp(m1 - m0))
    w1_ref[...] = 1.0 - w0_ref[...]

    oh0 = (lane == e0).astype(jnp.int32)
    oh1 = (lane == e1).astype(jnp.int32)
    cp = _incl_cumsum_ax0(oh0 + (oh1 << 13))
    c0 = cp & 0x1FFF
    c1 = cp >> 13
    total0 = c0[t_rows - 1:t_rows, :]
    total1 = c1[t_rows - 1:t_rows, :]
    counts = total0 + total1
    pt = (counts + (_TILE - 1)) // _TILE
    incl = pt
    s = 1
    while s < _E:
        incl = incl + jnp.concatenate(
            [jnp.zeros((1, s), jnp.int32), incl[:, :-s]], axis=1)
        s *= 2
    ts = incl - pt
    ao = ts * _TILE
    rank0 = jnp.sum(oh0 * c0, axis=1, keepdims=True) - 1
    rank1 = (jnp.sum(oh1 * c1, axis=1, keepdims=True) - 1
             + jnp.sum(oh1 * total0, axis=1, keepdims=True))
    pos0_ref[...] = jnp.sum(oh0 * ao, axis=1, keepdims=True) + rank0
    pos1_ref[...] = jnp.sum(oh1 * ao, axis=1, keepdims=True) + rank1
    rowi = jax.lax.broadcasted_iota(jnp.int32, (nt, _E), 0)
    te_ref[...] = jnp.sum((rowi >= ts).astype(jnp.int32), axis=1,
                          keepdims=True) - 1
    nu_ref[...] = jnp.sum(pt, axis=1, keepdims=True)


def _make_dispatch(T, D, rows):
    per_w = T // _NW
    ch = 64
    n_g = per_w // ch
    mesh = plsc.VectorSubcoreMesh(core_axis_name="c", subcore_axis_name="s")

    @functools.partial(
        pl.kernel, mesh=mesh,
        out_type=jax.ShapeDtypeStruct((rows, D), jnp.float32),
        scratch_types=[
            pltpu.VMEM((ch,), jnp.int32),
            pltpu.VMEM((ch, D), jnp.float32),
            pltpu.SemaphoreType.DMA,
        ])
    def _dispatch(x_hbm, p0_hbm, p1_hbm, xs_hbm, idx_v, rows_v, sem):
        wid = lax.axis_index("s") * 2 + lax.axis_index("c")
        for g in range(n_g):
            base = wid * per_w + g * ch
            pltpu.sync_copy(x_hbm.at[pl.ds(base, ch)], rows_v)
            pltpu.sync_copy(p0_hbm.at[pl.ds(base, ch)], idx_v)
            pltpu.async_copy(rows_v, xs_hbm.at[idx_v], sem).wait()
            pltpu.sync_copy(p1_hbm.at[pl.ds(base, ch)], idx_v)
            pltpu.async_copy(rows_v, xs_hbm.at[idx_v], sem).wait()

    return _dispatch


def _make_cgather(T, D, rows):
    per_w = T // _NW
    ch = 64
    n_g = per_w // ch
    mesh = plsc.VectorSubcoreMesh(core_axis_name="c", subcore_axis_name="s")

    @functools.partial(
        pl.kernel, mesh=mesh,
        out_type=[
            jax.ShapeDtypeStruct((T, D), jnp.float32),
            jax.ShapeDtypeStruct((T, D), jnp.float32),
        ],
        scratch_types=[
            pltpu.VMEM((ch,), jnp.int32),
            pltpu.VMEM((ch,), jnp.int32),
            pltpu.VMEM((ch, D), jnp.float32),
            pltpu.VMEM((ch, D), jnp.float32),
            pltpu.SemaphoreType.DMA,
        ])
    def _cgather(ys_hbm, p0_hbm, p1_hbm, y0_hbm, y1_hbm,
                 i0_v, i1_v, r0_v, r1_v, sem):
        wid = lax.axis_index("s") * 2 + lax.axis_index("c")
        for g in range(n_g):
            base = wid * per_w + g * ch
            pltpu.sync_copy(p0_hbm.at[pl.ds(base, ch)], i0_v)
            pltpu.sync_copy(p1_hbm.at[pl.ds(base, ch)], i1_v)
            pltpu.async_copy(ys_hbm.at[i0_v], r0_v, sem).wait()
            pltpu.sync_copy(r0_v, y0_hbm.at[pl.ds(base, ch)])
            pltpu.async_copy(ys_hbm.at[i1_v], r1_v, sem).wait()
            pltpu.sync_copy(r1_v, y1_hbm.at[pl.ds(base, ch)])

    return _cgather


def _wadd_body(y0_ref, y1_ref, w0_ref, w1_ref, out_ref):
    out_ref[...] = (y0_ref[...] * w0_ref[...] + y1_ref[...] * w1_ref[...])


def _gffn_body(te_ref, nu_ref, x_ref, w1_ref, b1_ref, w2_ref, b2_ref,
               out_ref):
    del te_ref
    i = pl.program_id(0)

    @pl.when(i < nu_ref[0])
    def _():
        x = x_ref[...].astype(jnp.bfloat16)
        h = jnp.dot(x, w1_ref[0].astype(jnp.bfloat16),
                    preferred_element_type=jnp.float32) + b1_ref[0]
        a = h[:, :_DFF]
        g = h[:, _DFF:]
        act = a * (g / (1.0 + jnp.exp(-g)))
        out_ref[...] = jnp.dot(act.astype(jnp.bfloat16),
                               w2_ref[0].astype(jnp.bfloat16),
                               preferred_element_type=jnp.float32) + b2_ref[0]


def kernel(x, router_W, router_temp, W1, b1, W2, b2):
    B, S, D = x.shape
    x_flat = x.reshape(-1, D)
    T = x_flat.shape[0]
    nt = (_K * T) // _TILE + _E
    rows = nt * _TILE

    pos0, pos1, w0, w1, te, nu = pl.pallas_call(
        _router_body,
        out_shape=[
            jax.ShapeDtypeStruct((T, 1), jnp.int32),
            jax.ShapeDtypeStruct((T, 1), jnp.int32),
            jax.ShapeDtypeStruct((T, 1), jnp.float32),
            jax.ShapeDtypeStruct((T, 1), jnp.float32),
            jax.ShapeDtypeStruct((nt, 1), jnp.int32),
            jax.ShapeDtypeStruct((1, 1), jnp.int32),
        ],
    )(x_flat, router_W, router_temp.reshape(1, 1))

    p0 = pos0.reshape(T)
    p1 = pos1.reshape(T)
    tile_expert = te.reshape(nt)

    x_sorted = _make_dispatch(T, D, rows)(x_flat, p0, p1)

    y_sorted = pl.pallas_call(
        _gffn_body,
        grid_spec=pltpu.PrefetchScalarGridSpec(
            num_scalar_prefetch=2,
            grid=(nt,),
            in_specs=[
                pl.BlockSpec((_TILE, D), lambda i, te, nu: (i, 0)),
                pl.BlockSpec((1, D, 2 * _DFF),
                             lambda i, te, nu: (te[i], 0, 0)),
                pl.BlockSpec((1, 1, 2 * _DFF),
                             lambda i, te, nu: (te[i], 0, 0)),
                pl.BlockSpec((1, _DFF, D), lambda i, te, nu: (te[i], 0, 0)),
                pl.BlockSpec((1, 1, D), lambda i, te, nu: (te[i], 0, 0)),
            ],
            out_specs=pl.BlockSpec((_TILE, D), lambda i, te, nu: (i, 0)),
        ),
        out_shape=jax.ShapeDtypeStruct((rows, D), jnp.float32),
        compiler_params=pltpu.CompilerParams(
            dimension_semantics=("arbitrary",)),
    )(tile_expert, nu.reshape(1), x_sorted, W1,
      b1.reshape(_E, 1, 2 * _DFF), W2, b2.reshape(_E, 1, D))

    y0, y1 = _make_cgather(T, D, rows)(y_sorted, p0, p1)
    tt = 1024
    out = pl.pallas_call(
        _wadd_body,
        grid=(T // tt,),
        in_specs=[
            pl.BlockSpec((tt, D), lambda i: (i, 0)),
            pl.BlockSpec((tt, D), lambda i: (i, 0)),
            pl.BlockSpec((tt, 1), lambda i: (i, 0)),
            pl.BlockSpec((tt, 1), lambda i: (i, 0)),
        ],
        out_specs=pl.BlockSpec((tt, D), lambda i: (i, 0)),
        out_shape=jax.ShapeDtypeStruct((T, D), jnp.float32),
    )(y0, y1, w0, w1)
    return out.reshape(B, S, D)


# speedup vs baseline: 2.7983x; 1.0028x over previous
import functools
import jax
import jax.numpy as jnp
from jax import lax
from jax.experimental import pallas as pl
from jax.experimental.pallas import tpu as pltpu
from jax.experimental.pallas import tpu_sc as plsc

_NW = 32

_E = 8
_K = 2
_D = 768
_DFF = 2048
_TILE = 256


def _incl_cumsum_ax0(v):
    s = 1
    while s < v.shape[0]:
        v = v + jnp.concatenate(
            [jnp.zeros((s, v.shape[1]), v.dtype), v[:-s]], axis=0)
        s *= 2
    return v


def _router_body(x_ref, rw_ref, temp_ref, pos0_ref, pos1_ref, w0_ref, w1_ref,
                 te_ref, nu_ref):
    nt = te_ref.shape[0]
    x = x_ref[...]
    logits = jax.lax.dot_general(
        x, rw_ref[...], (((1,), (1,)), ((), ())),
        preferred_element_type=jnp.float32)
    logits = logits / temp_ref[...]
    t_rows = logits.shape[0]
    lane = jax.lax.broadcasted_iota(jnp.int32, (t_rows, _E), 1)
    m0 = jnp.max(logits, axis=1, keepdims=True)
    e0 = jnp.min(jnp.where(logits == m0, lane, _E), axis=1, keepdims=True)
    masked = jnp.where(lane == e0, -jnp.inf, logits)
    m1 = jnp.max(masked, axis=1, keepdims=True)
    e1 = jnp.min(jnp.where(masked == m1, lane, _E), axis=1, keepdims=True)
    w0_ref[...] = 1.0 / (1.0 + jnp.exp(m1 - m0))
    w1_ref[...] = 1.0 - w0_ref[...]

    oh0 = (lane == e0).astype(jnp.int32)
    oh1 = (lane == e1).astype(jnp.int32)
    cp = _incl_cumsum_ax0(oh0 + (oh1 << 13))
    c0 = cp & 0x1FFF
    c1 = cp >> 13
    total0 = c0[t_rows - 1:t_rows, :]
    total1 = c1[t_rows - 1:t_rows, :]
    counts = total0 + total1
    pt = (counts + (_TILE - 1)) // _TILE
    incl = pt
    s = 1
    while s < _E:
        incl = incl + jnp.concatenate(
            [jnp.zeros((1, s), jnp.int32), incl[:, :-s]], axis=1)
        s *= 2
    ts = incl - pt
    ao = ts * _TILE
    rank0 = jnp.sum(oh0 * c0, axis=1, keepdims=True) - 1
    rank1 = (jnp.sum(oh1 * c1, axis=1, keepdims=True) - 1
             + jnp.sum(oh1 * total0, axis=1, keepdims=True))
    pos0_ref[...] = jnp.sum(oh0 * ao, axis=1, keepdims=True) + rank0
    pos1_ref[...] = jnp.sum(oh1 * ao, axis=1, keepdims=True) + rank1
    rowi = jax.lax.broadcasted_iota(jnp.int32, (nt, _E), 0)
    te_ref[...] = jnp.sum((rowi >= ts).astype(jnp.int32), axis=1,
                          keepdims=True) - 1
    nu_ref[...] = jnp.sum(pt, axis=1, keepdims=True)


def _make_dispatch(T, D, rows):
    per_w = T // _NW
    ch = 64
    n_g = per_w // ch
    mesh = plsc.VectorSubcoreMesh(core_axis_name="c", subcore_axis_name="s")

    @functools.partial(
        pl.kernel, mesh=mesh,
        out_type=[
            jax.ShapeDtypeStruct((rows, D), jnp.float32),
            jax.ShapeDtypeStruct((rows, 128), jnp.float32),
        ],
        scratch_types=[
            pltpu.VMEM((ch,), jnp.int32),
            pltpu.VMEM((ch, D), jnp.float32),
            pltpu.VMEM((ch, 128), jnp.float32),
            pltpu.SemaphoreType.DMA,
        ])
    def _dispatch(x_hbm, p0_hbm, p1_hbm, w0w_hbm, w1w_hbm, xs_hbm, ws_hbm,
                  idx_v, rows_v, w_v, sem):
        wid = lax.axis_index("s") * 2 + lax.axis_index("c")
        for g in range(n_g):
            base = wid * per_w + g * ch
            pltpu.sync_copy(x_hbm.at[pl.ds(base, ch)], rows_v)
            pltpu.sync_copy(p0_hbm.at[pl.ds(base, ch)], idx_v)
            pltpu.sync_copy(w0w_hbm.at[pl.ds(base, ch)], w_v)
            pltpu.async_copy(rows_v, xs_hbm.at[idx_v], sem).wait()
            pltpu.async_copy(w_v, ws_hbm.at[idx_v], sem).wait()
            pltpu.sync_copy(p1_hbm.at[pl.ds(base, ch)], idx_v)
            pltpu.sync_copy(w1w_hbm.at[pl.ds(base, ch)], w_v)
            pltpu.async_copy(rows_v, xs_hbm.at[idx_v], sem).wait()
            pltpu.async_copy(w_v, ws_hbm.at[idx_v], sem).wait()

    return _dispatch


def _make_cgather(T, D, rows):
    per_w = T // _NW
    ch = 64
    n_g = per_w // ch
    mesh = plsc.VectorSubcoreMesh(core_axis_name="c", subcore_axis_name="s")

    @functools.partial(
        pl.kernel, mesh=mesh,
        out_type=jax.ShapeDtypeStruct((T, D), jnp.float32),
        scratch_types=[
            pltpu.VMEM((ch,), jnp.int32),
            pltpu.VMEM((ch,), jnp.int32),
            pltpu.VMEM((ch, D), jnp.float32),
            pltpu.VMEM((ch, D), jnp.float32),
            pltpu.SemaphoreType.DMA,
        ])
    def _combine(ys_hbm, p0_hbm, p1_hbm, out_hbm,
                 i0_v, i1_v, r0_v, r1_v, sem):
        wid = lax.axis_index("s") * 2 + lax.axis_index("c")
        for g in range(n_g):
            base = wid * per_w + g * ch
            pltpu.sync_copy(p0_hbm.at[pl.ds(base, ch)], i0_v)
            pltpu.sync_copy(p1_hbm.at[pl.ds(base, ch)], i1_v)
            pltpu.async_copy(ys_hbm.at[i0_v], r0_v, sem).wait()
            pltpu.async_copy(ys_hbm.at[i1_v], r1_v, sem).wait()

            def t_body(t, c2):
                for c in range(D // 16):
                    r0_v[t, pl.ds(c * 16, 16)] = (
                        r0_v[t, pl.ds(c * 16, 16)]
                        + r1_v[t, pl.ds(c * 16, 16)])
                return c2

            lax.fori_loop(0, ch, t_body, 0)
            pltpu.sync_copy(r0_v, out_hbm.at[pl.ds(base, ch)])

    return _combine


def _gffn_body(te_ref, nu_ref, x_ref, ws_ref, w1_ref, b1_ref, w2_ref, b2_ref,
               out_ref):
    del te_ref
    i = pl.program_id(0)

    @pl.when(i < nu_ref[0])
    def _():
        x = x_ref[...].astype(jnp.bfloat16)
        h = jnp.dot(x, w1_ref[0].astype(jnp.bfloat16),
                    preferred_element_type=jnp.float32) + b1_ref[0]
        a = h[:, :_DFF]
        g = h[:, _DFF:]
        act = a * (g / (1.0 + jnp.exp(-g)))
        y = jnp.dot(act.astype(jnp.bfloat16), w2_ref[0].astype(jnp.bfloat16),
                    preferred_element_type=jnp.float32) + b2_ref[0]
        out_ref[...] = y * ws_ref[...][:, 0:1]


def kernel(x, router_W, router_temp, W1, b1, W2, b2):
    B, S, D = x.shape
    x_flat = x.reshape(-1, D)
    T = x_flat.shape[0]
    nt = (_K * T) // _TILE + _E
    rows = nt * _TILE

    pos0, pos1, w0, w1, te, nu = pl.pallas_call(
        _router_body,
        out_shape=[
            jax.ShapeDtypeStruct((T, 1), jnp.int32),
            jax.ShapeDtypeStruct((T, 1), jnp.int32),
            jax.ShapeDtypeStruct((T, 1), jnp.float32),
            jax.ShapeDtypeStruct((T, 1), jnp.float32),
            jax.ShapeDtypeStruct((nt, 1), jnp.int32),
            jax.ShapeDtypeStruct((1, 1), jnp.int32),
        ],
    )(x_flat, router_W, router_temp.reshape(1, 1))

    p0 = pos0.reshape(T)
    p1 = pos1.reshape(T)
    tile_expert = te.reshape(nt)

    w0w = jnp.broadcast_to(w0, (T, 128))
    w1w = jnp.broadcast_to(w1, (T, 128))
    x_sorted, w_sorted = _make_dispatch(T, D, rows)(x_flat, p0, p1, w0w, w1w)

    y_sorted = pl.pallas_call(
        _gffn_body,
        grid_spec=pltpu.PrefetchScalarGridSpec(
            num_scalar_prefetch=2,
            grid=(nt,),
            in_specs=[
                pl.BlockSpec((_TILE, D), lambda i, te, nu: (i, 0)),
                pl.BlockSpec((_TILE, 128), lambda i, te, nu: (i, 0)),
                pl.BlockSpec((1, D, 2 * _DFF),
                             lambda i, te, nu: (te[i], 0, 0)),
                pl.BlockSpec((1, 1, 2 * _DFF),
                             lambda i, te, nu: (te[i], 0, 0)),
                pl.BlockSpec((1, _DFF, D), lambda i, te, nu: (te[i], 0, 0)),
                pl.BlockSpec((1, 1, D), lambda i, te, nu: (te[i], 0, 0)),
            ],
            out_specs=pl.BlockSpec((_TILE, D), lambda i, te, nu: (i, 0)),
        ),
        out_shape=jax.ShapeDtypeStruct((rows, D), jnp.float32),
        compiler_params=pltpu.CompilerParams(
            dimension_semantics=("arbitrary",)),
    )(tile_expert, nu.reshape(1), x_sorted, w_sorted, W1,
      b1.reshape(_E, 1, 2 * _DFF), W2, b2.reshape(_E, 1, D))

    out = _make_cgather(T, D, rows)(y_sorted, p0, p1)
    return out.reshape(B, S, D)


# speedup vs baseline: 2.8228x; 1.0087x over previous
import functools
import jax
import jax.numpy as jnp
from jax import lax
from jax.experimental import pallas as pl
from jax.experimental.pallas import tpu as pltpu
from jax.experimental.pallas import tpu_sc as plsc

_NW = 32

_E = 8
_K = 2
_D = 768
_DFF = 2048
_TILE = 256


def _incl_cumsum_ax0(v):
    s = 1
    while s < v.shape[0]:
        v = v + jnp.concatenate(
            [jnp.zeros((s, v.shape[1]), v.dtype), v[:-s]], axis=0)
        s *= 2
    return v


def _router_body(x_ref, rw_ref, temp_ref, pos0_ref, pos1_ref, w0_ref, w1_ref,
                 te_ref, nu_ref):
    nt = te_ref.shape[0]
    x = x_ref[...]
    logits = jax.lax.dot_general(
        x, rw_ref[...], (((1,), (1,)), ((), ())),
        preferred_element_type=jnp.float32)
    logits = logits / temp_ref[...]
    t_rows = logits.shape[0]
    lane = jax.lax.broadcasted_iota(jnp.int32, (t_rows, _E), 1)
    m0 = jnp.max(logits, axis=1, keepdims=True)
    e0 = jnp.min(jnp.where(logits == m0, lane, _E), axis=1, keepdims=True)
    masked = jnp.where(lane == e0, -jnp.inf, logits)
    m1 = jnp.max(masked, axis=1, keepdims=True)
    e1 = jnp.min(jnp.where(masked == m1, lane, _E), axis=1, keepdims=True)
    w0_ref[...] = 1.0 / (1.0 + jnp.exp(m1 - m0))
    w1_ref[...] = 1.0 - w0_ref[...]

    oh0 = (lane == e0).astype(jnp.int32)
    oh1 = (lane == e1).astype(jnp.int32)
    cp = _incl_cumsum_ax0(oh0 + (oh1 << 13))
    c0 = cp & 0x1FFF
    c1 = cp >> 13
    total0 = c0[t_rows - 1:t_rows, :]
    total1 = c1[t_rows - 1:t_rows, :]
    counts = total0 + total1
    pt = (counts + (_TILE - 1)) // _TILE
    incl = pt
    s = 1
    while s < _E:
        incl = incl + jnp.concatenate(
            [jnp.zeros((1, s), jnp.int32), incl[:, :-s]], axis=1)
        s *= 2
    ts = incl - pt
    ao = ts * _TILE
    rank0 = jnp.sum(oh0 * c0, axis=1, keepdims=True) - 1
    rank1 = (jnp.sum(oh1 * c1, axis=1, keepdims=True) - 1
             + jnp.sum(oh1 * total0, axis=1, keepdims=True))
    pos0_ref[...] = jnp.sum(oh0 * ao, axis=1, keepdims=True) + rank0
    pos1_ref[...] = jnp.sum(oh1 * ao, axis=1, keepdims=True) + rank1
    rowi = jax.lax.broadcasted_iota(jnp.int32, (nt, _E), 0)
    te_ref[...] = jnp.sum((rowi >= ts).astype(jnp.int32), axis=1,
                          keepdims=True) - 1
    nu_ref[...] = jnp.sum(pt, axis=1, keepdims=True)


def _make_dispatch(T, D, rows):
    per_w = T // _NW
    ch = 64
    n_g = per_w // ch
    mesh = plsc.VectorSubcoreMesh(core_axis_name="c", subcore_axis_name="s")

    @functools.partial(
        pl.kernel, mesh=mesh,
        out_type=[
            jax.ShapeDtypeStruct((rows, D), jnp.float32),
            jax.ShapeDtypeStruct((rows, 128), jnp.float32),
        ],
        scratch_types=[
            pltpu.VMEM((ch,), jnp.int32),
            pltpu.VMEM((ch,), jnp.int32),
            pltpu.VMEM((ch, D), jnp.float32),
            pltpu.VMEM((ch, 128), jnp.float32),
            pltpu.VMEM((ch, 128), jnp.float32),
            pltpu.SemaphoreType.DMA,
        ])
    def _dispatch(x_hbm, p0_hbm, p1_hbm, w0w_hbm, w1w_hbm, xs_hbm, ws_hbm,
                  i0_v, i1_v, rows_v, w0_v, w1_v, sem):
        wid = lax.axis_index("s") * 2 + lax.axis_index("c")
        for g in range(n_g):
            base = wid * per_w + g * ch
            pltpu.sync_copy(x_hbm.at[pl.ds(base, ch)], rows_v)
            pltpu.sync_copy(p0_hbm.at[pl.ds(base, ch)], i0_v)
            pltpu.sync_copy(p1_hbm.at[pl.ds(base, ch)], i1_v)
            pltpu.sync_copy(w0w_hbm.at[pl.ds(base, ch)], w0_v)
            pltpu.sync_copy(w1w_hbm.at[pl.ds(base, ch)], w1_v)
            c0 = pltpu.async_copy(rows_v, xs_hbm.at[i0_v], sem)
            c1 = pltpu.async_copy(rows_v, xs_hbm.at[i1_v], sem)
            c2 = pltpu.async_copy(w0_v, ws_hbm.at[i0_v], sem)
            c3 = pltpu.async_copy(w1_v, ws_hbm.at[i1_v], sem)
            c0.wait()
            c1.wait()
            c2.wait()
            c3.wait()

    return _dispatch


def _make_cgather(T, D, rows):
    per_w = T // _NW
    ch = 64
    n_g = per_w // ch
    mesh = plsc.VectorSubcoreMesh(core_axis_name="c", subcore_axis_name="s")

    @functools.partial(
        pl.kernel, mesh=mesh,
        out_type=jax.ShapeDtypeStruct((T, D), jnp.float32),
        scratch_types=[
            pltpu.VMEM((ch,), jnp.int32),
            pltpu.VMEM((ch,), jnp.int32),
            pltpu.VMEM((ch, D), jnp.float32),
            pltpu.VMEM((ch, D), jnp.float32),
            pltpu.SemaphoreType.DMA,
        ])
    def _combine(ys_hbm, p0_hbm, p1_hbm, out_hbm,
                 i0_v, i1_v, r0_v, r1_v, sem):
        wid = lax.axis_index("s") * 2 + lax.axis_index("c")
        for g in range(n_g):
            base = wid * per_w + g * ch
            pltpu.sync_copy(p0_hbm.at[pl.ds(base, ch)], i0_v)
            pltpu.sync_copy(p1_hbm.at[pl.ds(base, ch)], i1_v)
            c0 = pltpu.async_copy(ys_hbm.at[i0_v], r0_v, sem)
            c1 = pltpu.async_copy(ys_hbm.at[i1_v], r1_v, sem)
            c0.wait()
            c1.wait()

            def t_body(t, c2):
                for c in range(D // 16):
                    r0_v[t, pl.ds(c * 16, 16)] = (
                        r0_v[t, pl.ds(c * 16, 16)]
                        + r1_v[t, pl.ds(c * 16, 16)])
                return c2

            lax.fori_loop(0, ch, t_body, 0)
            pltpu.sync_copy(r0_v, out_hbm.at[pl.ds(base, ch)])

    return _combine


def _gffn_body(te_ref, nu_ref, x_ref, ws_ref, w1_ref, b1_ref, w2_ref, b2_ref,
               out_ref):
    del te_ref
    i = pl.program_id(0)

    @pl.when(i < nu_ref[0])
    def _():
        x = x_ref[...].astype(jnp.bfloat16)
        h = jnp.dot(x, w1_ref[0].astype(jnp.bfloat16),
                    preferred_element_type=jnp.float32) + b1_ref[0]
        a = h[:, :_DFF]
        g = h[:, _DFF:]
        act = a * (g / (1.0 + jnp.exp(-g)))
        y = jnp.dot(act.astype(jnp.bfloat16), w2_ref[0].astype(jnp.bfloat16),
                    preferred_element_type=jnp.float32) + b2_ref[0]
        out_ref[...] = y * ws_ref[...][:, 0:1]


def kernel(x, router_W, router_temp, W1, b1, W2, b2):
    B, S, D = x.shape
    x_flat = x.reshape(-1, D)
    T = x_flat.shape[0]
    nt = (_K * T) // _TILE + _E
    rows = nt * _TILE

    pos0, pos1, w0, w1, te, nu = pl.pallas_call(
        _router_body,
        out_shape=[
            jax.ShapeDtypeStruct((T, 1), jnp.int32),
            jax.ShapeDtypeStruct((T, 1), jnp.int32),
            jax.ShapeDtypeStruct((T, 1), jnp.float32),
            jax.ShapeDtypeStruct((T, 1), jnp.float32),
            jax.ShapeDtypeStruct((nt, 1), jnp.int32),
            jax.ShapeDtypeStruct((1, 1), jnp.int32),
        ],
    )(x_flat, router_W, router_temp.reshape(1, 1))

    p0 = pos0.reshape(T)
    p1 = pos1.reshape(T)
    tile_expert = te.reshape(nt)

    w0w = jnp.broadcast_to(w0, (T, 128))
    w1w = jnp.broadcast_to(w1, (T, 128))
    x_sorted, w_sorted = _make_dispatch(T, D, rows)(x_flat, p0, p1, w0w, w1w)

    y_sorted = pl.pallas_call(
        _gffn_body,
        grid_spec=pltpu.PrefetchScalarGridSpec(
            num_scalar_prefetch=2,
            grid=(nt,),
            in_specs=[
                pl.BlockSpec((_TILE, D), lambda i, te, nu: (i, 0)),
                pl.BlockSpec((_TILE, 128), lambda i, te, nu: (i, 0)),
                pl.BlockSpec((1, D, 2 * _DFF),
                             lambda i, te, nu: (te[i], 0, 0)),
                pl.BlockSpec((1, 1, 2 * _DFF),
                             lambda i, te, nu: (te[i], 0, 0)),
                pl.BlockSpec((1, _DFF, D), lambda i, te, nu: (te[i], 0, 0)),
                pl.BlockSpec((1, 1, D), lambda i, te, nu: (te[i], 0, 0)),
            ],
            out_specs=pl.BlockSpec((_TILE, D), lambda i, te, nu: (i, 0)),
        ),
        out_shape=jax.ShapeDtypeStruct((rows, D), jnp.float32),
        compiler_params=pltpu.CompilerParams(
            dimension_semantics=("arbitrary",)),
    )(tile_expert, nu.reshape(1), x_sorted, w_sorted, W1,
      b1.reshape(_E, 1, 2 * _DFF), W2, b2.reshape(_E, 1, D))

    out = _make_cgather(T, D, rows)(y_sorted, p0, p1)
    return out.reshape(B, S, D)
